# baseline (speedup 1.0000x reference)
import functools

import jax
import jax.numpy as jnp
from jax import lax
from jax.experimental import pallas as pl
from jax.experimental.pallas import tpu as pltpu
from jax.experimental.pallas import tpu_sc as plsc

_VOCAB = 1000000
_EMB_DIM = 64
_KNOW_DIM = 32
_BATCH = 4096
_HIST = 50

_N = _BATCH * _HIST
_CHUNK = 128
_NW = 32
_PER_W = _N // _NW
_ROWS_W = _PER_W // _CHUNK


def _sc_body(x_hbm, word_hbm, know_hbm, out_hbm, idx_v, word_v, know_v,
             sem_w, sem_k):
    nc = 2
    wid = lax.axis_index("s") * nc + lax.axis_index("c")
    pltpu.sync_copy(x_hbm.at[pl.ds(wid * _ROWS_W, _ROWS_W)], idx_v)

    def step(j, carry):
        cw = pltpu.async_copy(word_hbm.at[idx_v.at[j]], word_v, sem_w)
        ck = pltpu.async_copy(know_hbm.at[idx_v.at[j]], know_v, sem_k)
        cw.wait()
        ck.wait()
        row0 = wid * _PER_W + j * _CHUNK
        pltpu.sync_copy(word_v, out_hbm.at[pl.ds(row0, _CHUNK),
                                           pl.ds(0, _EMB_DIM)])
        pltpu.sync_copy(know_v, out_hbm.at[pl.ds(row0, _CHUNK),
                                           pl.ds(_EMB_DIM, _KNOW_DIM)])
        return carry

    lax.fori_loop(0, _ROWS_W, step, 0)


@jax.jit
def _sc_lookup(x2d, word_table, knowledge_table):
    mesh = plsc.VectorSubcoreMesh(core_axis_name="c", subcore_axis_name="s")
    return pl.kernel(
        _sc_body,
        out_type=jax.ShapeDtypeStruct((_N, _EMB_DIM + _KNOW_DIM),
                                      jnp.float32),
        mesh=mesh,
        scratch_types=[
            pltpu.VMEM((_ROWS_W, _CHUNK), jnp.int32),
            pltpu.VMEM((_CHUNK, _EMB_DIM), jnp.float32),
            pltpu.VMEM((_CHUNK, _KNOW_DIM), jnp.float32),
            pltpu.SemaphoreType.DMA,
            pltpu.SemaphoreType.DMA,
        ],
        compiler_params=pltpu.CompilerParams(use_tc_tiling_on_sc=False),
    )(x2d, word_table, knowledge_table)


def kernel(x, word_table, knowledge_table):
    x2d = x.astype(jnp.int32).reshape(_N // _CHUNK, _CHUNK)
    out = _sc_lookup(x2d, word_table, knowledge_table)
    return out.reshape(_BATCH, _HIST, _EMB_DIM + _KNOW_DIM)
